# Initial kernel scaffold; baseline (speedup 1.0000x reference)
import jax
import jax.numpy as jnp
from jax.experimental import pallas as pl


def kernel(x, router_weight, router_bias, w1, w2, shared_w1, shared_w2):
    raise NotImplementedError("write your pallas kernel here")


# speedup vs baseline: 1.3675x; 1.3675x over previous
import functools

import jax
import jax.numpy as jnp
from jax import lax
from jax.experimental import pallas as pl
from jax.experimental.pallas import tpu as pltpu
from jax.experimental.pallas import tpu_sc as plsc

DIM = 2048
INTER = 1024
NE = 16
T = 128
BI = 256
NB = INTER // BI


def _router_body(x_ref, rw_ref, scores_ref):
    x = x_ref[...]
    rw = rw_ref[...]
    logits = lax.dot_general(
        x, rw, (((1,), (1,)), ((), ())),
        preferred_element_type=jnp.float32,
        precision=lax.Precision.HIGHEST,
    )
    scores_ref[...] = jax.nn.sigmoid(logits)


def _router_scores(x, router_weight, interpret=False):
    return pl.pallas_call(
        _router_body,
        out_shape=jax.ShapeDtypeStruct((T, NE), jnp.float32),
        interpret=interpret,
    )(x, router_weight)


_NC = 2
_NS = 16
_NW = _NC * _NS
_TPW = T // _NW


def _gate_body(scores_hbm, rb_hbm, gt_hbm, rb_v, sc_v, g_v):
    wid = lax.axis_index("s") * _NC + lax.axis_index("c")
    base = wid * _TPW
    pltpu.sync_copy(rb_hbm, rb_v)
    pltpu.sync_copy(scores_hbm.at[pl.ds(base, _TPW)], sc_v)
    rbv = rb_v[...]
    rbs = [rbv[e] for e in range(NE)]
    lanes = lax.iota(jnp.int32, NE)
    neg = jnp.float32(-3.0e38)
    for t in range(_TPW):
        s = sc_v[t, :]
        ss = [s[e] for e in range(NE)]
        m1, i1, raw1 = neg, jnp.int32(-1), jnp.float32(0.0)
        for e in range(NE):
            sele = ss[e] + rbs[e]
            b = sele > m1
            m1 = jnp.where(b, sele, m1)
            i1 = jnp.where(b, e, i1)
            raw1 = jnp.where(b, ss[e], raw1)
        m2, i2, raw2 = neg, jnp.int32(-1), jnp.float32(0.0)
        for e in range(NE):
            sele = ss[e] + rbs[e]
            b = (sele > m2) & (i1 != e)
            m2 = jnp.where(b, sele, m2)
            i2 = jnp.where(b, e, i2)
            raw2 = jnp.where(b, ss[e], raw2)
        denom = raw1 + raw2
        top = (jnp.where(lanes == i1, raw1, jnp.float32(0.0))
               + jnp.where(lanes == i2, raw2, jnp.float32(0.0)))
        g_v[t, :] = top / denom
    pltpu.sync_copy(g_v, gt_hbm.at[pl.ds(base, _TPW)])


def _gates(scores, router_bias):
    mesh = plsc.VectorSubcoreMesh(core_axis_name="c", subcore_axis_name="s")
    f = functools.partial(
        pl.kernel,
        out_type=jax.ShapeDtypeStruct((T, NE), jnp.float32),
        mesh=mesh,
        scratch_types=[
            pltpu.VMEM((NE,), jnp.float32),
            pltpu.VMEM((_TPW, NE), jnp.float32),
            pltpu.VMEM((_TPW, NE), jnp.float32),
        ],
    )(_gate_body)
    return f(scores, router_bias)


def _ffn_body(x_ref, g_ref, w1g_ref, w1u_ref, w2_ref,
              sw1g_ref, sw1u_ref, sw2_ref, out_ref):
    ee = pl.program_id(0)
    k = pl.program_id(1)
    is_shared = ee == 0

    @pl.when((ee == 0) & (k == 0))
    def _():
        out_ref[...] = jnp.zeros_like(out_ref)

    wg = jnp.where(is_shared, sw1g_ref[...], w1g_ref[0])
    wu = jnp.where(is_shared, sw1u_ref[...], w1u_ref[0])
    wd = jnp.where(is_shared, sw2_ref[...], w2_ref[0])
    xb = x_ref[...].astype(jnp.bfloat16)
    cdims = (((1,), (1,)), ((), ()))
    hg = lax.dot_general(xb, wg.astype(jnp.bfloat16), cdims,
                         preferred_element_type=jnp.float32)
    hu = lax.dot_general(xb, wu.astype(jnp.bfloat16), cdims,
                         preferred_element_type=jnp.float32)
    a = (hg * jax.nn.sigmoid(hg)) * hu
    g = jnp.where(is_shared, jnp.float32(1.0), g_ref[0, 0, :])
    a = a * g[:, None]
    y = lax.dot_general(a.astype(jnp.bfloat16), wd.astype(jnp.bfloat16),
                        cdims, preferred_element_type=jnp.float32)
    out_ref[...] += y


def _ffn(x, g, w1, w2, shared_w1, shared_w2, interpret=False):
    clamp = lambda e: jnp.maximum(e - 1, 0)
    kk = lambda e, k: jnp.where(e == 0, 0, k)
    sk = lambda e, k: jnp.where(e == 0, k, NB - 1)
    return pl.pallas_call(
        _ffn_body,
        grid=(NE + 1, NB),
        in_specs=[
            pl.BlockSpec((T, DIM), lambda e, k: (0, 0)),
            pl.BlockSpec((1, 1, T), lambda e, k: (clamp(e), 0, 0)),
            pl.BlockSpec((1, BI, DIM), lambda e, k: (clamp(e), kk(e, k), 0)),
            pl.BlockSpec((1, BI, DIM),
                         lambda e, k: (clamp(e), kk(e, k) + NB, 0)),
            pl.BlockSpec((1, DIM, BI), lambda e, k: (clamp(e), 0, kk(e, k))),
            pl.BlockSpec((BI, DIM), lambda e, k: (sk(e, k), 0)),
            pl.BlockSpec((BI, DIM), lambda e, k: (sk(e, k) + NB, 0)),
            pl.BlockSpec((DIM, BI), lambda e, k: (0, sk(e, k))),
        ],
        out_specs=pl.BlockSpec((T, DIM), lambda e, k: (0, 0)),
        out_shape=jax.ShapeDtypeStruct((T, DIM), jnp.float32),
        interpret=interpret,
    )(x, g, w1, w1, w2, shared_w1, shared_w1, shared_w2)


def kernel(x, router_weight, router_bias, w1, w2, shared_w1, shared_w2):
    scores = _router_scores(x, router_weight)
    gt = _gates(scores, router_bias)
    g = gt.T.reshape(NE, 1, T)
    return _ffn(x, g, w1, w2, shared_w1, shared_w2)
